# baseline (speedup 1.0000x reference)
import functools

import jax
import jax.numpy as jnp
from jax.experimental import pallas as pl


def _body(x_ref, edge_ref, tok_ref, o_ref, *, blk, half, n):
    j = pl.program_id(1)
    nblk = half // blk
    tok = tok_ref[...][None, None, None, :]

    @pl.when(j != nblk - 1)
    def _():
        o_ref[:, 0:1] = jnp.broadcast_to(tok, (1, 1, blk, o_ref.shape[3]))
        o_ref[:, 1:2] = x_ref[...]

    @pl.when(j == nblk - 1)
    def _():
        rows = j * blk + jax.lax.broadcasted_iota(
            jnp.int32, (1, 1, blk, 1), 2
        )
        reg0 = jnp.where(
            rows == half - 1, edge_ref[...][:, :, None, :], tok
        )
        reg1 = jnp.where(rows + half <= n - 2, x_ref[...], tok)
        o_ref[:, 0:1] = reg0
        o_ref[:, 1:2] = reg1


def kernel(x, token):
    B, N, D = x.shape
    half = N // 2
    BLK = 512
    nblk = half // BLK

    x4 = x.reshape(B, 2, half, D)
    edge = jax.lax.slice_in_dim(x, half - 1, half, axis=1).reshape(B, 1, D)

    out = pl.pallas_call(
        functools.partial(_body, blk=BLK, half=half, n=N),
        grid=(B, nblk),
        in_specs=[
            pl.BlockSpec((1, 1, BLK, D), lambda b, j: (b, 1, j, 0)),
            pl.BlockSpec((1, 1, D), lambda b, j: (b, 0, 0)),
            pl.BlockSpec((D,), lambda b, j: (0,)),
        ],
        out_specs=pl.BlockSpec((1, 2, BLK, D), lambda b, j: (b, 0, j, 0)),
        out_shape=jax.ShapeDtypeStruct((B, 2, half, D), x.dtype),
    )(x4, edge, token)
    return out.reshape(B, N, D)


# speedup vs baseline: 1.0623x; 1.0623x over previous
import functools

import jax
import jax.numpy as jnp
from jax.experimental import pallas as pl


def _body(x_ref, edge_ref, tok_ref, o_ref, *, blk, half, n):
    j = pl.program_id(1)
    nblk = half // blk
    tok = tok_ref[...][None, None, None, :]

    @pl.when(j != nblk - 1)
    def _():
        o_ref[:, 0:1] = jnp.broadcast_to(tok, (1, 1, blk, o_ref.shape[3]))
        o_ref[:, 1:2] = x_ref[...]

    @pl.when(j == nblk - 1)
    def _():
        rows = j * blk + jax.lax.broadcasted_iota(
            jnp.int32, (1, 1, blk, 1), 2
        )
        reg0 = jnp.where(
            rows == half - 1, edge_ref[...][:, :, None, :], tok
        )
        reg1 = jnp.where(rows + half <= n - 2, x_ref[...], tok)
        o_ref[:, 0:1] = reg0
        o_ref[:, 1:2] = reg1


def kernel(x, token):
    B, N, D = x.shape
    half = N // 2
    BLK = 1024
    nblk = half // BLK

    x4 = x.reshape(B, 2, half, D)
    edge = jax.lax.slice_in_dim(x, half - 1, half, axis=1).reshape(B, 1, D)

    out = pl.pallas_call(
        functools.partial(_body, blk=BLK, half=half, n=N),
        grid=(B, nblk),
        in_specs=[
            pl.BlockSpec((1, 1, BLK, D), lambda b, j: (b, 1, j, 0)),
            pl.BlockSpec((1, 1, D), lambda b, j: (b, 0, 0)),
            pl.BlockSpec((D,), lambda b, j: (0,)),
        ],
        out_specs=pl.BlockSpec((1, 2, BLK, D), lambda b, j: (b, 0, j, 0)),
        out_shape=jax.ShapeDtypeStruct((B, 2, half, D), x.dtype),
    )(x4, edge, token)
    return out.reshape(B, N, D)


# speedup vs baseline: 1.0941x; 1.0299x over previous
import functools

import jax
import jax.numpy as jnp
from jax.experimental import pallas as pl


def _body(x_ref, edge_ref, tok_ref, o_ref, *, blk, half, n):
    j = pl.program_id(1)
    nblk = half // blk
    tok = tok_ref[...][None, None, None, :]

    @pl.when(j != nblk - 1)
    def _():
        o_ref[:, 0:1] = jnp.broadcast_to(tok, (1, 1, blk, o_ref.shape[3]))
        o_ref[:, 1:2] = x_ref[...]

    @pl.when(j == nblk - 1)
    def _():
        rows = j * blk + jax.lax.broadcasted_iota(
            jnp.int32, (1, 1, blk, 1), 2
        )
        reg0 = jnp.where(
            rows == half - 1, edge_ref[...][:, :, None, :], tok
        )
        reg1 = jnp.where(rows + half <= n - 2, x_ref[...], tok)
        o_ref[:, 0:1] = reg0
        o_ref[:, 1:2] = reg1


def kernel(x, token):
    B, N, D = x.shape
    half = N // 2
    BLK = 2048
    nblk = half // BLK

    x4 = x.reshape(B, 2, half, D)
    edge = jax.lax.slice_in_dim(x, half - 1, half, axis=1).reshape(B, 1, D)

    out = pl.pallas_call(
        functools.partial(_body, blk=BLK, half=half, n=N),
        grid=(B, nblk),
        in_specs=[
            pl.BlockSpec((1, 1, BLK, D), lambda b, j: (b, 1, j, 0)),
            pl.BlockSpec((1, 1, D), lambda b, j: (b, 0, 0)),
            pl.BlockSpec((D,), lambda b, j: (0,)),
        ],
        out_specs=pl.BlockSpec((1, 2, BLK, D), lambda b, j: (b, 0, j, 0)),
        out_shape=jax.ShapeDtypeStruct((B, 2, half, D), x.dtype),
    )(x4, edge, token)
    return out.reshape(B, N, D)
